# Initial kernel scaffold; baseline (speedup 1.0000x reference)
import jax
import jax.numpy as jnp
from jax.experimental import pallas as pl


def kernel(factor_potentials, edge_index):
    raise NotImplementedError("write your pallas kernel here")


# speedup vs baseline: 47.2720x; 47.2720x over previous
import functools

import jax
import jax.numpy as jnp
from jax import lax
from jax.experimental import pallas as pl
from jax.experimental.pallas import tpu as pltpu
from jax.experimental.pallas import tpu_sc as plsc

F = 800000
V = 100000
TC_COLS = 128
TC_ROWS = F // TC_COLS
TC_BLK = 256
TC_GRID = (TC_ROWS + TC_BLK - 1) // TC_BLK
SC_COLS = 125
SC_ROWS = F // SC_COLS
CHUNK = 40
NS = 16
NC = 2
V_SLICE = 6256

_f32 = jnp.float32


def _lse2(x, y):
    m = jnp.maximum(x, y)
    return m + jnp.log1p(jnp.exp(-jnp.abs(x - y)))


def _tc0_body(p00, p01, p10, p11, fa, fb):
    a0, a1, b0, b1 = p00[...], p01[...], p10[...], p11[...]
    fa[...] = 0.5 * (_lse2(b0, b1) - _lse2(a0, a1))
    fb[...] = 0.5 * (_lse2(a1, b1) - _lse2(a0, b0))


def _tc_step_body(ga, gb, fa, fb, wa, wb, p00, p01, p10, p11,
                  wa_o, wb_o, fa_o, fb_o):
    fav, fbv = fa[...], fb[...]
    wan = 0.5 * (ga[...] - fav) + 0.5 * wa[...]
    wbn = 0.5 * (gb[...] - fbv) + 0.5 * wb[...]
    a0, a1, b0, b1 = p00[...], p01[...], p10[...], p11[...]
    wa_o[...] = wan
    wb_o[...] = wbn
    fa_o[...] = 0.5 * (_lse2(b0, b1 + wbn) - _lse2(a0, a1 + wbn)) + 0.5 * fav
    fb_o[...] = 0.5 * (_lse2(a1, b1 + wan) - _lse2(a0, b0 + wan)) + 0.5 * fbv


def _tc_fb_body(p00, p01, p10, p11, wa, wb, s1, s2):
    i = pl.program_id(0)
    rows = lax.broadcasted_iota(jnp.int32, (TC_BLK, TC_COLS), 0) + i * TC_BLK
    msk = rows < TC_ROWS
    z = jnp.zeros((TC_BLK, TC_COLS), _f32)
    a0 = jnp.where(msk, p00[...], z)
    a1 = jnp.where(msk, p01[...], z)
    b0 = jnp.where(msk, p10[...], z)
    b1 = jnp.where(msk, p11[...], z)
    wav = jnp.where(msk, wa[...], z)
    wbv = jnp.where(msk, wb[...], z)
    t00 = a0
    t01 = a1 + wbv
    t10 = b0 + wav
    t11 = b1 + wav + wbv
    m = jnp.maximum(jnp.maximum(t00, t01), jnp.maximum(t10, t11))
    e00 = jnp.exp(t00 - m)
    e01 = jnp.exp(t01 - m)
    e10 = jnp.exp(t10 - m)
    e11 = jnp.exp(t11 - m)
    lz = jnp.log(e00 + e01 + e10 + e11)
    zq = m + lz
    e00 = jnp.exp(t00 - zq)
    e01 = jnp.exp(t01 - zq)
    e10 = jnp.exp(t10 - zq)
    e11 = jnp.exp(t11 - zq)
    s1_blk = jnp.sum(jnp.where(msk, e00 * a0 + e01 * a1 + e10 * b0 + e11 * b1, z))
    s2_blk = jnp.sum(jnp.where(
        msk,
        e00 * (t00 - zq) + e01 * (t01 - zq) + e10 * (t10 - zq) + e11 * (t11 - zq),
        z))

    @pl.when(i == 0)
    def _():
        s1[...] = jnp.reshape(s1_blk, (1, 1))
        s2[...] = jnp.reshape(s2_blk, (1, 1))

    @pl.when(i > 0)
    def _():
        s1[...] += jnp.reshape(s1_blk, (1, 1))
        s2[...] += jnp.reshape(s2_blk, (1, 1))


def _tc_var_body(vbd, deg, s3):
    d = vbd[...]
    e = jnp.exp(-jnp.abs(d))
    sig = jnp.where(d >= 0, 1.0 / (1.0 + e), e / (1.0 + e))
    sp = jnp.maximum(d, 0.0) + jnp.log1p(e)
    inner = sig * d - sp
    s3[...] = jnp.reshape(jnp.sum((deg[...] - 1.0) * inner), (1, 1))


def _ew_spec():
    return pl.BlockSpec((TC_BLK, TC_COLS), lambda i: (i, 0))


def _scalar_spec():
    return pl.BlockSpec((1, 1), lambda i: (0, 0))


_tc0 = pl.pallas_call(
    _tc0_body,
    grid=(TC_GRID,),
    in_specs=[_ew_spec()] * 4,
    out_specs=[_ew_spec()] * 2,
    out_shape=[jax.ShapeDtypeStruct((TC_ROWS, TC_COLS), _f32)] * 2,
)

_tc_step = pl.pallas_call(
    _tc_step_body,
    grid=(TC_GRID,),
    in_specs=[_ew_spec()] * 10,
    out_specs=[_ew_spec()] * 4,
    out_shape=[jax.ShapeDtypeStruct((TC_ROWS, TC_COLS), _f32)] * 4,
)

_tc_fb = pl.pallas_call(
    _tc_fb_body,
    grid=(TC_GRID,),
    in_specs=[_ew_spec()] * 6,
    out_specs=[_scalar_spec()] * 2,
    out_shape=[jax.ShapeDtypeStruct((1, 1), _f32)] * 2,
)

_VPAD = 782 * 128
_tc_var = pl.pallas_call(
    _tc_var_body,
    out_shape=jax.ShapeDtypeStruct((1, 1), _f32),
)


def _sc_sg_body(ia, ib, fa, fb, ga, gb, vbd, table, idxv, msgv, gv, zbuf):
    cid = lax.axis_index("c")
    sid = lax.axis_index("s")
    start = jnp.minimum(sid * V_SLICE, V - V_SLICE)

    def zloop(i, carry):
        zbuf[pl.ds(i * 16, 16)] = jnp.zeros((16,), _f32)
        return carry

    lax.fori_loop(0, V_SLICE // 16, zloop, 0)
    pltpu.sync_copy(zbuf, table.at[pl.ds(start, V_SLICE)])
    plsc.subcore_barrier()

    rows_per_tile = SC_ROWS // NS

    def scatter_pass(idx_hbm, msg_hbm):
        def chunk(i, carry):
            base = sid * rows_per_tile + i * CHUNK
            pltpu.sync_copy(idx_hbm.at[pl.ds(base, CHUNK)], idxv)
            pltpu.sync_copy(msg_hbm.at[pl.ds(base, CHUNK)], msgv)
            for j in range(CHUNK):
                pltpu.sync_copy(msgv.at[j], table.at[idxv.at[j]], add=True)
            return carry

        lax.fori_loop(0, rows_per_tile // CHUNK, chunk, 0)

    scatter_pass(ia, fa)
    scatter_pass(ib, fb)
    plsc.subcore_barrier()

    @pl.when(cid == 0)
    def _():
        pltpu.sync_copy(table.at[pl.ds(start, V_SLICE)], zbuf)
        pltpu.sync_copy(zbuf, vbd.at[pl.ds(start, V_SLICE)])

    wid = cid * NS + sid
    rows_per_wtile = SC_ROWS // (NC * NS)

    def gather_pass(idx_hbm, out_hbm):
        def chunk(i, carry):
            base = wid * rows_per_wtile + i * CHUNK
            pltpu.sync_copy(idx_hbm.at[pl.ds(base, CHUNK)], idxv)
            for j in range(CHUNK):
                pltpu.sync_copy(table.at[idxv.at[j]], gv.at[j])
            pltpu.sync_copy(gv, out_hbm.at[pl.ds(base, CHUNK)])
            return carry

        lax.fori_loop(0, rows_per_wtile // CHUNK, chunk, 0)

    gather_pass(ia, ga)
    gather_pass(ib, gb)


_sc_sg = pl.kernel(
    _sc_sg_body,
    out_type=[
        jax.ShapeDtypeStruct((SC_ROWS, SC_COLS), _f32),
        jax.ShapeDtypeStruct((SC_ROWS, SC_COLS), _f32),
        jax.ShapeDtypeStruct((V,), _f32),
    ],
    mesh=plsc.VectorSubcoreMesh(
        core_axis_name="c", subcore_axis_name="s", num_cores=NC, num_subcores=NS),
    scratch_types=[
        pltpu.VMEM_SHARED((V,), _f32),
        pltpu.VMEM((CHUNK, SC_COLS), jnp.int32),
        pltpu.VMEM((CHUNK, SC_COLS), _f32),
        pltpu.VMEM((CHUNK, SC_COLS), _f32),
        pltpu.VMEM((V_SLICE,), _f32),
    ],
)


def _sc_deg_body(ia, ib, deg, table, idxv, onesv, zbuf):
    cid = lax.axis_index("c")
    sid = lax.axis_index("s")
    start = jnp.minimum(sid * V_SLICE, V - V_SLICE)
    for j in range(8):
        onesv[pl.ds(j * 16, 16)] = jnp.ones((16,), _f32)

    @pl.when(cid == 0)
    def _():
        def zloop(i, carry):
            zbuf[pl.ds(i * 16, 16)] = jnp.zeros((16,), _f32)
            return carry

        lax.fori_loop(0, V_SLICE // 16, zloop, 0)
        pltpu.sync_copy(zbuf, table.at[pl.ds(start, V_SLICE)])

    plsc.subcore_barrier()

    rows_per_tile = SC_ROWS // NS

    @pl.when(cid == 0)
    def _():
        def scatter_pass(idx_hbm):
            def chunk(i, carry):
                base = sid * rows_per_tile + i * CHUNK
                pltpu.sync_copy(idx_hbm.at[pl.ds(base, CHUNK)], idxv)
                for j in range(CHUNK):
                    pltpu.sync_copy(onesv.at[pl.ds(0, SC_COLS)],
                                    table.at[idxv.at[j]], add=True)
                return carry

            lax.fori_loop(0, rows_per_tile // CHUNK, chunk, 0)

        scatter_pass(ia)
        scatter_pass(ib)

    plsc.subcore_barrier()

    @pl.when(cid == 0)
    def _():
        pltpu.sync_copy(table.at[pl.ds(start, V_SLICE)], zbuf)
        pltpu.sync_copy(zbuf, deg.at[pl.ds(start, V_SLICE)])


_sc_deg = pl.kernel(
    _sc_deg_body,
    out_type=jax.ShapeDtypeStruct((V,), _f32),
    mesh=plsc.VectorSubcoreMesh(
        core_axis_name="c", subcore_axis_name="s", num_cores=NC, num_subcores=NS),
    scratch_types=[
        pltpu.VMEM_SHARED((V,), _f32),
        pltpu.VMEM((CHUNK, SC_COLS), jnp.int32),
        pltpu.VMEM((128,), _f32),
        pltpu.VMEM((V_SLICE,), _f32),
    ],
)


def kernel(factor_potentials, edge_index):
    fp = factor_potentials.astype(_f32)
    p00 = fp[:, 0].reshape(TC_ROWS, TC_COLS)
    p01 = fp[:, 1].reshape(TC_ROWS, TC_COLS)
    p10 = fp[:, 2].reshape(TC_ROWS, TC_COLS)
    p11 = fp[:, 3].reshape(TC_ROWS, TC_COLS)
    ia2 = edge_index[0].reshape(SC_ROWS, SC_COLS)
    ib2 = edge_index[1].reshape(SC_ROWS, SC_COLS)
    zeros = jnp.zeros((V,), _f32)

    fa, fb = _tc0(p00, p01, p10, p11)
    deg = _sc_deg(ia2, ib2)

    wa = jnp.zeros((TC_ROWS, TC_COLS), _f32)
    wb = jnp.zeros((TC_ROWS, TC_COLS), _f32)
    vbd = zeros
    for _k in range(5):
        ga2, gb2, vbd = _sc_sg(
            ia2, ib2, fa.reshape(SC_ROWS, SC_COLS), fb.reshape(SC_ROWS, SC_COLS))
        ga = ga2.reshape(TC_ROWS, TC_COLS)
        gb = gb2.reshape(TC_ROWS, TC_COLS)
        wa, wb, fa, fb = _tc_step(ga, gb, fa, fb, wa, wb, p00, p01, p10, p11)

    s1, s2 = _tc_fb(p00, p01, p10, p11, wa, wb)
    vbd_p = jnp.concatenate([vbd, jnp.zeros((_VPAD - V,), _f32)]).reshape(782, 128)
    deg_p = jnp.concatenate([deg, jnp.ones((_VPAD - V,), _f32)]).reshape(782, 128)
    s3 = _tc_var(vbd_p, deg_p)
    return (-s1[0, 0] + s2[0, 0] - s3[0, 0]).astype(_f32)
